# baseline (speedup 1.0000x reference)
import functools
import jax
import jax.numpy as jnp
from jax import lax
from jax.experimental import pallas as pl
from jax.experimental.pallas import tpu as pltpu, tpu_sc as plsc


def _tc_bcast(w_ref, out_ref):
    out_ref[...] = jnp.broadcast_to(w_ref[...][None, :, :], out_ref.shape)


def _tc_copies(W, nb, S, H):
    BS = 512
    return pl.pallas_call(
        _tc_bcast,
        grid=(S // BS,),
        in_specs=[pl.BlockSpec((BS, H), lambda i: (i, 0))],
        out_specs=pl.BlockSpec((nb, BS, H), lambda i: (0, i, 0)),
        out_shape=jax.ShapeDtypeStruct((nb, S, H), W.dtype),
    )(W)


def _sc_copies(W, nb, S, H):
    info = plsc.get_sparse_core_info()
    NW = info.num_cores * info.num_subcores
    rows_per_w = S // NW
    CH = 32
    n_chunks = rows_per_w // CH
    mesh = plsc.VectorSubcoreMesh(core_axis_name="c", subcore_axis_name="s")

    @functools.partial(
        pl.kernel, mesh=mesh,
        out_type=jax.ShapeDtypeStruct((nb, S, H), W.dtype),
        scratch_types=[
            pltpu.VMEM((2, CH, H), W.dtype),
            pltpu.SemaphoreType.DMA,
            pltpu.SemaphoreType.DMA,
        ],
    )
    def body(w_hbm, out_hbm, buf, in_sem, out_sem):
        wid = lax.axis_index("s") * info.num_cores + lax.axis_index("c")
        base = wid * rows_per_w
        pltpu.async_copy(w_hbm.at[pl.ds(base, CH)], buf.at[0], in_sem).wait()
        for c in range(n_chunks):
            slot = c % 2
            off = base + c * CH
            if c + 1 < n_chunks:
                nxt = pltpu.async_copy(
                    w_hbm.at[pl.ds(off + CH, CH)], buf.at[1 - slot], in_sem)
            writes = [
                pltpu.async_copy(buf.at[slot],
                                 out_hbm.at[b, pl.ds(off, CH)], out_sem)
                for b in range(nb)
            ]
            for wcp in writes:
                wcp.wait()
            if c + 1 < n_chunks:
                nxt.wait()

    return body(W)


def kernel(x, W):
    B, S, H = x.shape
    NB_SC = 1
    NB_TC = B - NB_SC
    tc = _tc_copies(W[:S], NB_TC, S, H)
    sc = _sc_copies(W[:S], NB_SC, S, H)
    return jnp.concatenate([tc, sc], axis=0)


# speedup vs baseline: 2.9697x; 2.9697x over previous
import jax
import jax.numpy as jnp
from jax.experimental import pallas as pl
from jax.experimental.pallas import tpu as pltpu


def _fanout_body(w_ref, out_ref, sem):
    i = pl.program_id(0)
    nb = out_ref.shape[0]
    bs = w_ref.shape[0]
    copies = [
        pltpu.make_async_copy(w_ref, out_ref.at[b, pl.ds(i * bs, bs)], sem)
        for b in range(nb)
    ]
    for c in copies:
        c.start()
    for c in copies:
        c.wait()


def kernel(x, W):
    B, S, H = x.shape
    BS = 512
    return pl.pallas_call(
        _fanout_body,
        grid=(S // BS,),
        in_specs=[pl.BlockSpec((BS, H), lambda i: (i, 0))],
        out_specs=pl.BlockSpec(memory_space=pl.ANY),
        out_shape=jax.ShapeDtypeStruct((B, S, H), W.dtype),
        scratch_shapes=[pltpu.SemaphoreType.DMA],
    )(W[:S])


# speedup vs baseline: 3.4536x; 1.1629x over previous
import jax
import jax.numpy as jnp
from jax.experimental import pallas as pl
from jax.experimental.pallas import tpu as pltpu

_BS = 512
_NBUF = 3


def _pipeline_body(w_hbm, out_hbm, buf, in_sems, out_sems):
    nb = out_hbm.shape[0]
    n = w_hbm.shape[0] // _BS
    pending = [None] * _NBUF

    def start_in(i, slot):
        cp = pltpu.make_async_copy(
            w_hbm.at[pl.ds(i * _BS, _BS)], buf.at[slot], in_sems.at[slot])
        cp.start()
        return cp

    def start_writes(i, slot):
        cps = [
            pltpu.make_async_copy(
                buf.at[slot], out_hbm.at[b, pl.ds(i * _BS, _BS)],
                out_sems.at[slot])
            for b in range(nb)
        ]
        for c in cps:
            c.start()
        return cps

    def drain(cps):
        if cps:
            for c in cps:
                c.wait()

    in_flight = [None] * _NBUF
    in_flight[0] = start_in(0, 0)
    for i in range(n):
        slot = i % _NBUF
        if i + 1 < n:
            nslot = (i + 1) % _NBUF
            drain(pending[nslot])
            pending[nslot] = None
            in_flight[nslot] = start_in(i + 1, nslot)
        in_flight[slot].wait()
        pending[slot] = start_writes(i, slot)
    for cps in pending:
        drain(cps)


def kernel(x, W):
    B, S, H = x.shape
    return pl.pallas_call(
        _pipeline_body,
        in_specs=[pl.BlockSpec(memory_space=pl.ANY)],
        out_specs=pl.BlockSpec(memory_space=pl.ANY),
        out_shape=jax.ShapeDtypeStruct((B, S, H), W.dtype),
        scratch_shapes=[
            pltpu.VMEM((_NBUF, _BS, H), W.dtype),
            pltpu.SemaphoreType.DMA((_NBUF,)),
            pltpu.SemaphoreType.DMA((_NBUF,)),
        ],
    )(W[:S])


# speedup vs baseline: 3.6337x; 1.0521x over previous
import jax
import jax.numpy as jnp
from jax.experimental import pallas as pl
from jax.experimental.pallas import tpu as pltpu

_BS = 1024
_NBUF = 3


def _pipeline_body(w_hbm, out_hbm, buf, in_sems, out_sems):
    nb = out_hbm.shape[0]
    n = w_hbm.shape[0] // _BS
    pending = [None] * _NBUF

    def start_in(i, slot):
        cp = pltpu.make_async_copy(
            w_hbm.at[pl.ds(i * _BS, _BS)], buf.at[slot], in_sems.at[slot])
        cp.start()
        return cp

    def start_writes(i, slot):
        cps = [
            pltpu.make_async_copy(
                buf.at[slot], out_hbm.at[b, pl.ds(i * _BS, _BS)],
                out_sems.at[slot])
            for b in range(nb)
        ]
        for c in cps:
            c.start()
        return cps

    def drain(cps):
        if cps:
            for c in cps:
                c.wait()

    in_flight = [None] * _NBUF
    in_flight[0] = start_in(0, 0)
    for i in range(n):
        slot = i % _NBUF
        if i + 1 < n:
            nslot = (i + 1) % _NBUF
            drain(pending[nslot])
            pending[nslot] = None
            in_flight[nslot] = start_in(i + 1, nslot)
        in_flight[slot].wait()
        pending[slot] = start_writes(i, slot)
    for cps in pending:
        drain(cps)


def kernel(x, W):
    B, S, H = x.shape
    return pl.pallas_call(
        _pipeline_body,
        in_specs=[pl.BlockSpec(memory_space=pl.ANY)],
        out_specs=pl.BlockSpec(memory_space=pl.ANY),
        out_shape=jax.ShapeDtypeStruct((B, S, H), W.dtype),
        scratch_shapes=[
            pltpu.VMEM((_NBUF, _BS, H), W.dtype),
            pltpu.SemaphoreType.DMA((_NBUF,)),
            pltpu.SemaphoreType.DMA((_NBUF,)),
        ],
    )(W[:S])
